# Initial kernel scaffold; baseline (speedup 1.0000x reference)
import jax
import jax.numpy as jnp
from jax.experimental import pallas as pl


def kernel(logits, value):
    raise NotImplementedError("write your pallas kernel here")


# speedup vs baseline: 1.6291x; 1.6291x over previous
import functools

import jax
import jax.numpy as jnp
from jax import lax
from jax.experimental import pallas as pl
from jax.experimental.pallas import tpu as pltpu
from jax.experimental.pallas import tpu_sc as plsc

A_DIM = 26
N_CLASSES = 100000
BATCH = 4096

_NC = 2
_NS = 16
_L = 16


_W = 8192
_G = -(-N_CLASSES // _W)


def _lse_body(x_ref, o_ref, m_ref, s_ref):
    g = pl.program_id(0)

    @pl.when(g == 0)
    def _init():
        m_ref[...] = jnp.full_like(m_ref, -jnp.inf)
        s_ref[...] = jnp.zeros_like(s_ref)

    x = x_ref[...]
    col = g * _W + lax.broadcasted_iota(jnp.int32, x.shape, 1)
    x = jnp.where(col < N_CLASSES, x, -jnp.inf)
    bm = jnp.max(x, axis=1, keepdims=True)
    m_old = m_ref[...]
    m_new = jnp.maximum(m_old, bm)
    s_new = s_ref[...] * jnp.exp(m_old - m_new) + jnp.sum(
        jnp.exp(x - m_new), axis=1, keepdims=True
    )
    m_ref[...] = m_new
    s_ref[...] = s_new

    @pl.when(g == _G - 1)
    def _fin():
        o_ref[0, 0] = jnp.sum(m_new + jnp.log(s_new))


def _lse_const(logits):
    return pl.pallas_call(
        _lse_body,
        grid=(_G,),
        in_specs=[pl.BlockSpec((A_DIM, _W), lambda g: (0, g))],
        out_specs=pl.BlockSpec(memory_space=pltpu.SMEM),
        out_shape=jax.ShapeDtypeStruct((1, 1), jnp.float32),
        scratch_shapes=[
            pltpu.VMEM((A_DIM, 1), jnp.float32),
            pltpu.VMEM((A_DIM, 1), jnp.float32),
        ],
    )(logits)


_RB = BATCH // 128


def _sc_body(logits_hbm, valt_hbm, out_hbm, row_v, idx_v, acc_v, sidx_v, shared):
    c = lax.axis_index("c")
    s = lax.axis_index("s")
    wid = s * _NC + c

    @pl.when(wid < A_DIM)
    def _gather():
        pltpu.sync_copy(logits_hbm.at[wid], row_v)
        pltpu.sync_copy(valt_hbm.at[wid], idx_v)

        def outer(r, carry):
            def inner(k, carry2):
                idx = idx_v[r, pl.ds(k * _L, _L)]
                acc_v[r, pl.ds(k * _L, _L)] = plsc.load_gather(row_v, [idx])
                return carry2

            return lax.fori_loop(0, 128 // _L, inner, carry)

        lax.fori_loop(0, _RB, outer, 0)

    iota = lax.broadcasted_iota(jnp.int32, (_L,), 0)
    sidx_v[pl.ds(0, _L)] = iota
    sidx_v[pl.ds(_L, _L)] = iota + _L

    plsc.subcore_barrier()

    @pl.when(s == 0)
    def _seed():
        pltpu.sync_copy(acc_v, shared)

    plsc.subcore_barrier()

    @pl.when((s >= 1) & (wid < A_DIM))
    def _accum():
        pltpu.sync_copy(acc_v, shared.at[sidx_v], add=True)

    plsc.subcore_barrier()

    @pl.when(s == 0)
    def _out():
        pltpu.sync_copy(shared, out_hbm.at[c])


@functools.partial(jax.jit, static_argnums=())
def _sc_gather(logits, valt):
    mesh = plsc.VectorSubcoreMesh(
        core_axis_name="c", subcore_axis_name="s", num_cores=_NC, num_subcores=_NS
    )
    f = pl.kernel(
        _sc_body,
        out_type=jax.ShapeDtypeStruct((_NC, _RB, 128), jnp.float32),
        mesh=mesh,
        scratch_types=[
            pltpu.VMEM((N_CLASSES,), jnp.float32),
            pltpu.VMEM((_RB, 128), jnp.int32),
            pltpu.VMEM((_RB, 128), jnp.float32),
            pltpu.VMEM((2 * _L,), jnp.int32),
            pltpu.VMEM_SHARED((_RB, 128), jnp.float32),
        ],
        compiler_params=pltpu.CompilerParams(needs_layout_passes=False),
    )
    return f(logits, valt)


def kernel(logits, value):
    valt = value.T.reshape(A_DIM, _RB, 128)
    c = _lse_const(logits)
    parts = _sc_gather(logits, valt)
    gsum = (parts[0] + parts[1]).reshape(BATCH)
    return gsum - c[0, 0]
